# baseline (speedup 1.0000x reference)
import functools

import jax
import jax.numpy as jnp
from jax import lax
from jax.experimental import pallas as pl
from jax.experimental.pallas import tpu as pltpu
from jax.experimental.pallas import tpu_sc as plsc


@functools.lru_cache(maxsize=None)
def _build_hist_kernel(e, n, n_pad, nc, ns):
    mesh = plsc.VectorSubcoreMesh(core_axis_name="c", subcore_axis_name="s",
                                  num_cores=nc, num_subcores=ns)
    nw = nc * ns
    chunk = n_pad // ns
    ew = 8 * (e // (nw * 8))
    tail = e - nw * ew
    ew16 = -16 * (-ew // 16)
    cut = n - (ns - 1) * chunk
    padlen = chunk - cut
    assert tail % 16 == 0 and chunk % 16 == 0
    assert 0 < cut <= chunk and cut % 8 == 0

    scratch = [
        pltpu.VMEM((ew,), jnp.int32),
        pltpu.VMEM((ew16,), jnp.float32),
        pltpu.VMEM((chunk,), jnp.float32),
        pltpu.VMEM_SHARED((n_pad,), jnp.float32),
        pltpu.SemaphoreType.DMA,
    ]
    if tail:
        scratch.append(pltpu.VMEM((tail,), jnp.int32))

    @functools.partial(
        pl.kernel,
        out_type=(jax.ShapeDtypeStruct((nc * n,), jnp.float32),
                  jax.ShapeDtypeStruct((nc * padlen,), jnp.float32)),
        mesh=mesh,
        scratch_types=scratch,
    )
    def hist_kernel(idx_hbm, out_hbm, pad_hbm, idx_v, ones_v, hbuf_v,
                    hist_sh, idx_sem, *tail_v):
        c = lax.axis_index("c")
        s = lax.axis_index("s")
        wid = s * nc + c

        idx_cp = pltpu.async_copy(idx_hbm.at[pl.ds(wid * ew, ew)], idx_v,
                                  idx_sem)

        def obody(i, carry):
            ones_v[pl.ds(i * 16, 16)] = jnp.ones((16,), jnp.float32)
            return carry

        lax.fori_loop(0, ew16 // 16, obody, jnp.int32(0))

        def zbody(i, carry):
            hbuf_v[pl.ds(i * 16, 16)] = jnp.zeros((16,), jnp.float32)
            return carry

        lax.fori_loop(0, chunk // 16, zbody, jnp.int32(0))
        pltpu.sync_copy(hbuf_v, hist_sh.at[pl.ds(s * chunk, chunk)])

        plsc.subcore_barrier()
        idx_cp.wait()

        pltpu.sync_copy(ones_v.at[pl.ds(0, ew)], hist_sh.at[idx_v],
                        add=True)

        if tail:
            @pl.when(wid == nw - 1)
            def _():
                pltpu.sync_copy(idx_hbm.at[pl.ds(nw * ew, tail)], tail_v[0])
                pltpu.sync_copy(ones_v.at[pl.ds(0, tail)],
                                hist_sh.at[tail_v[0]], add=True)

        plsc.subcore_barrier()

        pltpu.sync_copy(hist_sh.at[pl.ds(s * chunk, chunk)], hbuf_v)

        @pl.when(s < ns - 1)
        def _():
            pltpu.sync_copy(hbuf_v, out_hbm.at[pl.ds(c * n + s * chunk,
                                                     chunk)])

        @pl.when(s == ns - 1)
        def _():
            pltpu.sync_copy(hbuf_v.at[pl.ds(0, cut)],
                            out_hbm.at[pl.ds(c * n + (ns - 1) * chunk, cut)])
            if padlen:
                pltpu.sync_copy(hbuf_v.at[pl.ds(cut, padlen)],
                                pad_hbm.at[pl.ds(c * padlen, padlen)])

    return hist_kernel


@functools.lru_cache(maxsize=None)
def _build_sc_matvec_kernel(n, d, n_tc, n_sc, nc, ns):
    mesh = plsc.VectorSubcoreMesh(core_axis_name="c", subcore_axis_name="s",
                                  num_cores=nc, num_subcores=ns)
    nw = nc * ns
    rows_full = 16 * (-(-n_sc // (nw * 16)))
    rows_last = n_sc - (nw - 1) * rows_full
    rows16 = rows_full
    nvec = d // 16
    assert 0 < rows_last <= rows_full and rows_last % 16 == 0

    @functools.partial(
        pl.kernel,
        out_type=jax.ShapeDtypeStruct((nw * d,), jnp.float32),
        mesh=mesh,
        scratch_types=[
            pltpu.VMEM((rows_full * d,), jnp.float32),
            pltpu.VMEM((rows16,), jnp.float32),
            pltpu.VMEM((rows16,), jnp.float32),
            pltpu.VMEM((rows16,), jnp.float32),
            pltpu.VMEM((d,), jnp.float32),
            pltpu.SemaphoreType.DMA,
        ],
    )
    def scmv_kernel(w_hbm, f_hbm, out_hbm, f_v, w0_v, w1_v, ws_v, y_v,
                    f_sem):
        c = lax.axis_index("c")
        s = lax.axis_index("s")
        wid = s * nc + c
        row0 = n_tc + wid * rows_full
        last = wid == nw - 1

        @pl.when(jnp.logical_not(last))
        def _():
            pltpu.async_copy(f_hbm.at[pl.ds(row0 * d, rows_full * d)],
                             f_v.at[pl.ds(0, rows_full * d)], f_sem)
            pltpu.sync_copy(w_hbm.at[pl.ds(row0, rows_full)],
                            w0_v.at[pl.ds(0, rows_full)])
            pltpu.sync_copy(w_hbm.at[pl.ds(n + row0, rows_full)],
                            w1_v.at[pl.ds(0, rows_full)])

        @pl.when(last)
        def _():
            pltpu.async_copy(f_hbm.at[pl.ds(row0 * d, rows_last * d)],
                             f_v.at[pl.ds(0, rows_last * d)], f_sem)
            pltpu.sync_copy(w_hbm.at[pl.ds(row0, rows_last)],
                            w0_v.at[pl.ds(0, rows_last)])
            pltpu.sync_copy(w_hbm.at[pl.ds(n + row0, rows_last)],
                            w1_v.at[pl.ds(0, rows_last)])

        def wbody(i, carry):
            ws_v[pl.ds(i * 16, 16)] = (w0_v[pl.ds(i * 16, 16)] +
                                       w1_v[pl.ds(i * 16, 16)])
            return carry

        lax.fori_loop(0, rows16 // 16, wbody, jnp.int32(0))

        def gbody(g, acc):
            wvec = ws_v[pl.ds(g * 16, 16)]
            accl = list(acc)
            for t in range(16):
                wr = lax.gather(
                    wvec, jnp.full((16, 1), t, jnp.int32),
                    lax.GatherDimensionNumbers(
                        offset_dims=(), collapsed_slice_dims=(0,),
                        start_index_map=(0,)),
                    (1,), mode=lax.GatherScatterMode.PROMISE_IN_BOUNDS)
                off = (g * 16 + t) * d
                for j in range(nvec):
                    accl[j] = accl[j] + f_v[pl.ds(off + j * 16, 16)] * wr
            return tuple(accl)

        def run(nr):
            acc = lax.fori_loop(
                0, nr // 16, gbody,
                tuple(jnp.zeros((16,), jnp.float32) for _ in range(nvec)))
            for j in range(nvec):
                y_v[pl.ds(j * 16, 16)] = acc[j]

        @pl.when(jnp.logical_not(last))
        def _():
            pltpu.make_async_copy(f_hbm.at[pl.ds(row0 * d, rows_full * d)],
                                  f_v.at[pl.ds(0, rows_full * d)],
                                  f_sem).wait()
            run(rows_full)

        @pl.when(last)
        def _():
            pltpu.make_async_copy(f_hbm.at[pl.ds(row0 * d, rows_last * d)],
                                  f_v.at[pl.ds(0, rows_last * d)],
                                  f_sem).wait()
            run(rows_last)

        pltpu.sync_copy(y_v, out_hbm.at[pl.ds(wid * d, d)])

    return scmv_kernel


def _matvec_body(nk, kb8, w0_ref, w1_ref, f_ref, o_ref):
    k = pl.program_id(0)

    @pl.when(k == 0)
    def _():
        o_ref[...] = jnp.zeros_like(o_ref)

    w = w0_ref[0, 0] + w1_ref[0, 0]
    acc = o_ref[...]
    for r in range(8):
        acc += jnp.dot(w[r:r + 1], f_ref[pl.ds(r * kb8, kb8), :],
                       preferred_element_type=jnp.float32)
    o_ref[...] = acc


def kernel(u, neighs, features):
    del u
    e = neighs.shape[0]
    n, d = features.shape

    info = plsc.get_sparse_core_info()
    nc, ns = info.num_cores, info.num_subcores
    n_pad = ns * 128 * (-(-n // (ns * 128)))
    hist, _ = _build_hist_kernel(e, n, n_pad, nc, ns)(
        neighs.astype(jnp.int32))

    kb = next(b for b in (10000, 5000, 4000, 2500, 2000, 1000, 500, 8)
              if n % b == 0 and b % 8 == 0)
    nk = n // kb
    kb8 = kb // 8
    nk_sc = 2 if nk >= 5 else 0
    nk_tc = nk - nk_sc
    n_tc = nk_tc * kb

    w = hist.reshape(nc, nk, 8, kb8)
    out = pl.pallas_call(
        functools.partial(_matvec_body, nk_tc, kb8),
        grid=(nk_tc,),
        in_specs=[
            pl.BlockSpec((1, 1, 8, kb8), lambda k: (0, k, 0, 0)),
            pl.BlockSpec((1, 1, 8, kb8), lambda k: (1, k, 0, 0)),
            pl.BlockSpec((kb, d), lambda k: (k, 0)),
        ],
        out_specs=pl.BlockSpec((1, d), lambda k: (0, 0)),
        out_shape=jax.ShapeDtypeStruct((1, d), jnp.float32),
    )(w, w, features)
    h = out.reshape(d)

    if nk_sc:
        y_sc = _build_sc_matvec_kernel(n, d, n_tc, n - n_tc, nc, ns)(
            hist, features.reshape(n * d))
        h = h + jnp.sum(y_sc.reshape(nc * ns, d), axis=0)
    return h * (1.0 / e)


# speedup vs baseline: 1.0005x; 1.0005x over previous
import functools

import jax
import jax.numpy as jnp
from jax import lax
from jax.experimental import pallas as pl
from jax.experimental.pallas import tpu as pltpu
from jax.experimental.pallas import tpu_sc as plsc


@functools.lru_cache(maxsize=None)
def _build_hist_kernel(e, n, n_pad, nc, ns):
    mesh = plsc.VectorSubcoreMesh(core_axis_name="c", subcore_axis_name="s",
                                  num_cores=nc, num_subcores=ns)
    nw = nc * ns
    chunk = n_pad // ns
    ew = 8 * (e // (nw * 8))
    tail = e - nw * ew
    ew16 = -16 * (-ew // 16)
    cut = n - (ns - 1) * chunk
    padlen = chunk - cut
    assert tail % 16 == 0 and chunk % 16 == 0
    assert 0 < cut <= chunk and cut % 8 == 0

    scratch = [
        pltpu.VMEM((ew,), jnp.int32),
        pltpu.VMEM((ew16,), jnp.float32),
        pltpu.VMEM((chunk,), jnp.float32),
        pltpu.VMEM_SHARED((n_pad,), jnp.float32),
        pltpu.SemaphoreType.DMA,
    ]
    if tail:
        scratch.append(pltpu.VMEM((tail,), jnp.int32))

    @functools.partial(
        pl.kernel,
        out_type=(jax.ShapeDtypeStruct((nc * n,), jnp.float32),
                  jax.ShapeDtypeStruct((nc * padlen,), jnp.float32)),
        mesh=mesh,
        scratch_types=scratch,
    )
    def hist_kernel(idx_hbm, out_hbm, pad_hbm, idx_v, ones_v, hbuf_v,
                    hist_sh, idx_sem, *tail_v):
        c = lax.axis_index("c")
        s = lax.axis_index("s")
        wid = s * nc + c

        idx_cp = pltpu.async_copy(idx_hbm.at[pl.ds(wid * ew, ew)], idx_v,
                                  idx_sem)

        def obody(i, carry):
            ones_v[pl.ds(i * 16, 16)] = jnp.ones((16,), jnp.float32)
            return carry

        lax.fori_loop(0, ew16 // 16, obody, jnp.int32(0))

        def zbody(i, carry):
            hbuf_v[pl.ds(i * 16, 16)] = jnp.zeros((16,), jnp.float32)
            return carry

        lax.fori_loop(0, chunk // 16, zbody, jnp.int32(0))
        pltpu.sync_copy(hbuf_v, hist_sh.at[pl.ds(s * chunk, chunk)])

        plsc.subcore_barrier()
        idx_cp.wait()

        pltpu.sync_copy(ones_v.at[pl.ds(0, ew)], hist_sh.at[idx_v],
                        add=True)

        if tail:
            @pl.when(wid == nw - 1)
            def _():
                pltpu.sync_copy(idx_hbm.at[pl.ds(nw * ew, tail)], tail_v[0])
                pltpu.sync_copy(ones_v.at[pl.ds(0, tail)],
                                hist_sh.at[tail_v[0]], add=True)

        plsc.subcore_barrier()

        pltpu.sync_copy(hist_sh.at[pl.ds(s * chunk, chunk)], hbuf_v)

        @pl.when(s < ns - 1)
        def _():
            pltpu.sync_copy(hbuf_v, out_hbm.at[pl.ds(c * n + s * chunk,
                                                     chunk)])

        @pl.when(s == ns - 1)
        def _():
            pltpu.sync_copy(hbuf_v.at[pl.ds(0, cut)],
                            out_hbm.at[pl.ds(c * n + (ns - 1) * chunk, cut)])
            if padlen:
                pltpu.sync_copy(hbuf_v.at[pl.ds(cut, padlen)],
                                pad_hbm.at[pl.ds(c * padlen, padlen)])

    return hist_kernel


@functools.lru_cache(maxsize=None)
def _build_sc_matvec_kernel(n, d, n_tc, n_sc, nc, ns):
    mesh = plsc.VectorSubcoreMesh(core_axis_name="c", subcore_axis_name="s",
                                  num_cores=nc, num_subcores=ns)
    nw = nc * ns
    rows_full = 16 * (-(-n_sc // (nw * 16)))
    rows_last = n_sc - (nw - 1) * rows_full
    rows16 = rows_full
    nvec = d // 16
    assert 0 < rows_last <= rows_full and rows_last % 16 == 0

    @functools.partial(
        pl.kernel,
        out_type=jax.ShapeDtypeStruct((nw * d,), jnp.float32),
        mesh=mesh,
        scratch_types=[
            pltpu.VMEM((rows_full * d,), jnp.float32),
            pltpu.VMEM((rows16,), jnp.float32),
            pltpu.VMEM((rows16,), jnp.float32),
            pltpu.VMEM((rows16,), jnp.float32),
            pltpu.VMEM((d,), jnp.float32),
            pltpu.SemaphoreType.DMA,
        ],
    )
    def scmv_kernel(w_hbm, f_hbm, out_hbm, f_v, w0_v, w1_v, ws_v, y_v,
                    f_sem):
        c = lax.axis_index("c")
        s = lax.axis_index("s")
        wid = s * nc + c
        row0 = n_tc + wid * rows_full
        last = wid == nw - 1

        @pl.when(jnp.logical_not(last))
        def _():
            pltpu.async_copy(f_hbm.at[pl.ds(row0 * d, rows_full * d)],
                             f_v.at[pl.ds(0, rows_full * d)], f_sem)
            pltpu.sync_copy(w_hbm.at[pl.ds(row0, rows_full)],
                            w0_v.at[pl.ds(0, rows_full)])
            pltpu.sync_copy(w_hbm.at[pl.ds(n + row0, rows_full)],
                            w1_v.at[pl.ds(0, rows_full)])

        @pl.when(last)
        def _():
            pltpu.async_copy(f_hbm.at[pl.ds(row0 * d, rows_last * d)],
                             f_v.at[pl.ds(0, rows_last * d)], f_sem)
            pltpu.sync_copy(w_hbm.at[pl.ds(row0, rows_last)],
                            w0_v.at[pl.ds(0, rows_last)])
            pltpu.sync_copy(w_hbm.at[pl.ds(n + row0, rows_last)],
                            w1_v.at[pl.ds(0, rows_last)])

        def wbody(i, carry):
            ws_v[pl.ds(i * 16, 16)] = (w0_v[pl.ds(i * 16, 16)] +
                                       w1_v[pl.ds(i * 16, 16)])
            return carry

        lax.fori_loop(0, rows16 // 16, wbody, jnp.int32(0))

        def gbody(g, acc):
            wvec = ws_v[pl.ds(g * 16, 16)]
            accl = list(acc)
            for t in range(16):
                wr = lax.gather(
                    wvec, jnp.full((16, 1), t, jnp.int32),
                    lax.GatherDimensionNumbers(
                        offset_dims=(), collapsed_slice_dims=(0,),
                        start_index_map=(0,)),
                    (1,), mode=lax.GatherScatterMode.PROMISE_IN_BOUNDS)
                off = (g * 16 + t) * d
                for j in range(nvec):
                    accl[j] = accl[j] + f_v[pl.ds(off + j * 16, 16)] * wr
            return tuple(accl)

        def run(nr):
            acc = lax.fori_loop(
                0, nr // 16, gbody,
                tuple(jnp.zeros((16,), jnp.float32) for _ in range(nvec)))
            for j in range(nvec):
                y_v[pl.ds(j * 16, 16)] = acc[j]

        @pl.when(jnp.logical_not(last))
        def _():
            pltpu.make_async_copy(f_hbm.at[pl.ds(row0 * d, rows_full * d)],
                                  f_v.at[pl.ds(0, rows_full * d)],
                                  f_sem).wait()
            run(rows_full)

        @pl.when(last)
        def _():
            pltpu.make_async_copy(f_hbm.at[pl.ds(row0 * d, rows_last * d)],
                                  f_v.at[pl.ds(0, rows_last * d)],
                                  f_sem).wait()
            run(rows_last)

        pltpu.sync_copy(y_v, out_hbm.at[pl.ds(wid * d, d)])

    return scmv_kernel


def _matvec_body(nk, kb8, w0_ref, w1_ref, f_ref, o_ref):
    k = pl.program_id(0)

    @pl.when(k == 0)
    def _():
        o_ref[...] = jnp.zeros_like(o_ref)

    w = w0_ref[0, 0] + w1_ref[0, 0]
    acc = o_ref[...]
    for r in range(8):
        acc += jnp.dot(w[r:r + 1], f_ref[pl.ds(r * kb8, kb8), :],
                       preferred_element_type=jnp.float32)
    o_ref[...] = acc


def kernel(u, neighs, features):
    del u
    e = neighs.shape[0]
    n, d = features.shape

    info = plsc.get_sparse_core_info()
    nc, ns = info.num_cores, info.num_subcores
    n_pad = ns * 128 * (-(-n // (ns * 128)))
    hist, _ = _build_hist_kernel(e, n, n_pad, nc, ns)(
        neighs.astype(jnp.int32))

    kb = next(b for b in (10000, 5000, 4000, 2500, 2000, 1000, 500, 8)
              if n % b == 0 and b % 8 == 0)
    nk = n // kb
    kb8 = kb // 8
    nk_sc = 2 if nk >= 5 else 0
    nk_tc = nk - nk_sc
    n_tc = nk_tc * kb

    if nk_sc:
        y_sc = _build_sc_matvec_kernel(n, d, n_tc, n - n_tc, nc, ns)(
            hist, features.reshape(n * d))

    w = hist.reshape(nc, nk, 8, kb8)
    out = pl.pallas_call(
        functools.partial(_matvec_body, nk_tc, kb8),
        grid=(nk_tc,),
        in_specs=[
            pl.BlockSpec((1, 1, 8, kb8), lambda k: (0, k, 0, 0)),
            pl.BlockSpec((1, 1, 8, kb8), lambda k: (1, k, 0, 0)),
            pl.BlockSpec((kb, d), lambda k: (k, 0)),
        ],
        out_specs=pl.BlockSpec((1, d), lambda k: (0, 0)),
        out_shape=jax.ShapeDtypeStruct((1, d), jnp.float32),
    )(w, w, features)
    h = out.reshape(d)

    if nk_sc:
        h = h + jnp.sum(y_sc.reshape(nc * ns, d), axis=0)
    return h * (1.0 / e)


# speedup vs baseline: 1.0410x; 1.0405x over previous
import functools

import jax
import jax.numpy as jnp
from jax import lax
from jax.experimental import pallas as pl
from jax.experimental.pallas import tpu as pltpu
from jax.experimental.pallas import tpu_sc as plsc


@functools.lru_cache(maxsize=None)
def _build_hist_kernel(e, n, n_pad, nc, ns):
    mesh = plsc.VectorSubcoreMesh(core_axis_name="c", subcore_axis_name="s",
                                  num_cores=nc, num_subcores=ns)
    nw = nc * ns
    chunk = n_pad // ns
    ew = 8 * (e // (nw * 8))
    tail = e - nw * ew
    ew16 = -16 * (-ew // 16)
    cut = n - (ns - 1) * chunk
    padlen = chunk - cut
    assert tail % 16 == 0 and chunk % 16 == 0
    assert 0 < cut <= chunk and cut % 8 == 0

    scratch = [
        pltpu.VMEM((ew,), jnp.int32),
        pltpu.VMEM((ew16,), jnp.float32),
        pltpu.VMEM((chunk,), jnp.float32),
        pltpu.VMEM_SHARED((n_pad,), jnp.float32),
        pltpu.SemaphoreType.DMA,
    ]
    if tail:
        scratch.append(pltpu.VMEM((tail,), jnp.int32))

    @functools.partial(
        pl.kernel,
        out_type=(jax.ShapeDtypeStruct((nc * n,), jnp.float32),
                  jax.ShapeDtypeStruct((nc * padlen,), jnp.float32)),
        mesh=mesh,
        scratch_types=scratch,
    )
    def hist_kernel(idx_hbm, out_hbm, pad_hbm, idx_v, ones_v, hbuf_v,
                    hist_sh, idx_sem, *tail_v):
        c = lax.axis_index("c")
        s = lax.axis_index("s")
        wid = s * nc + c

        idx_cp = pltpu.async_copy(idx_hbm.at[pl.ds(wid * ew, ew)], idx_v,
                                  idx_sem)

        def obody(i, carry):
            ones_v[pl.ds(i * 16, 16)] = jnp.ones((16,), jnp.float32)
            return carry

        lax.fori_loop(0, ew16 // 16, obody, jnp.int32(0))

        def zbody(i, carry):
            hbuf_v[pl.ds(i * 16, 16)] = jnp.zeros((16,), jnp.float32)
            return carry

        lax.fori_loop(0, chunk // 16, zbody, jnp.int32(0))
        pltpu.sync_copy(hbuf_v, hist_sh.at[pl.ds(s * chunk, chunk)])

        plsc.subcore_barrier()
        idx_cp.wait()

        pltpu.sync_copy(ones_v.at[pl.ds(0, ew)], hist_sh.at[idx_v],
                        add=True)

        if tail:
            @pl.when(wid == nw - 1)
            def _():
                pltpu.sync_copy(idx_hbm.at[pl.ds(nw * ew, tail)], tail_v[0])
                pltpu.sync_copy(ones_v.at[pl.ds(0, tail)],
                                hist_sh.at[tail_v[0]], add=True)

        plsc.subcore_barrier()

        pltpu.sync_copy(hist_sh.at[pl.ds(s * chunk, chunk)], hbuf_v)

        @pl.when(s < ns - 1)
        def _():
            pltpu.sync_copy(hbuf_v, out_hbm.at[pl.ds(c * n + s * chunk,
                                                     chunk)])

        @pl.when(s == ns - 1)
        def _():
            pltpu.sync_copy(hbuf_v.at[pl.ds(0, cut)],
                            out_hbm.at[pl.ds(c * n + (ns - 1) * chunk, cut)])
            if padlen:
                pltpu.sync_copy(hbuf_v.at[pl.ds(cut, padlen)],
                                pad_hbm.at[pl.ds(c * padlen, padlen)])

    return hist_kernel


def _matvec_body(nk, kb8, inv_e, w0_ref, w1_ref, f_ref, o_ref):
    k = pl.program_id(0)

    @pl.when(k == 0)
    def _():
        o_ref[...] = jnp.zeros_like(o_ref)

    w = w0_ref[0, 0] + w1_ref[0, 0]
    acc = o_ref[...]
    for r in range(8):
        acc += jnp.dot(w[r:r + 1], f_ref[pl.ds(r * kb8, kb8), :],
                       preferred_element_type=jnp.float32)
    o_ref[...] = acc

    @pl.when(k == nk - 1)
    def _():
        o_ref[...] = o_ref[...] * inv_e


def kernel(u, neighs, features):
    del u
    e = neighs.shape[0]
    n, d = features.shape

    info = plsc.get_sparse_core_info()
    nc, ns = info.num_cores, info.num_subcores
    n_pad = ns * 128 * (-(-n // (ns * 128)))
    hist, _ = _build_hist_kernel(e, n, n_pad, nc, ns)(
        neighs.astype(jnp.int32))

    kb = next(b for b in (20000, 10000, 5000, 4000, 2500, 2000, 1000, 500, 8)
              if n % b == 0 and b % 8 == 0)
    nk = n // kb
    kb8 = kb // 8
    w = hist.reshape(nc, nk, 8, kb8)
    out = pl.pallas_call(
        functools.partial(_matvec_body, nk, kb8, 1.0 / e),
        grid=(nk,),
        in_specs=[
            pl.BlockSpec((1, 1, 8, kb8), lambda k: (0, k, 0, 0)),
            pl.BlockSpec((1, 1, 8, kb8), lambda k: (1, k, 0, 0)),
            pl.BlockSpec((kb, d), lambda k: (k, 0)),
        ],
        out_specs=pl.BlockSpec((1, d), lambda k: (0, 0)),
        out_shape=jax.ShapeDtypeStruct((1, d), jnp.float32),
    )(w, w, features)
    return out.reshape(d)
